# Initial kernel scaffold; baseline (speedup 1.0000x reference)
import jax
import jax.numpy as jnp
from jax.experimental import pallas as pl


def kernel(rows, shifts, latent, coords, base_values, W0, b0, W1, b1, W2, b2, W3, b3, Wd, bd, ctf):
    raise NotImplementedError("write your pallas kernel here")


# speedup vs baseline: 51.4991x; 51.4991x over previous
import functools

import numpy as np
import jax
import jax.numpy as jnp
from jax import lax
from jax.experimental import pallas as pl
from jax.experimental.pallas import tpu as pltpu
from jax.experimental.pallas import tpu_sc as plsc

BATCH = 16
XSIZE = 128
NPIX = XSIZE * XSIZE
ACC = NPIX + 192
W0_FIRST = 30.0

CH = 5120
NCHUNK = 30
HALF = CH * NCHUNK
VPAD = 2 * HALF
VB = 6144
NW = 32

_HIGH = jax.lax.Precision.HIGHEST


def _stage_a_body(rs_ref, latent_ref, w0_ref, w1_ref, w2_ref, w3_ref,
                  bias_ref, aux_ref, wd_ref,
                  idx_ref, va_ref, vb_ref, vc_ref, vd_ref):
    rs = rs_ref[...]
    a = rs[:, 0:1]
    b = rs[:, 1:2]
    g = rs[:, 2:3]
    ca, sa = jnp.cos(a), jnp.sin(a)
    cb, sb = jnp.cos(b), jnp.sin(b)
    cg, sg = jnp.cos(g), jnp.sin(g)
    r00 = cg * cb * ca - sg * sa
    r01 = cg * cb * sa + sg * ca
    r02 = -cg * sb
    r10 = -sg * cb * ca - cg * sa
    r11 = -sg * cb * sa + cg * ca
    r12 = sg * sb
    sx = rs[:, 3:4]
    sy = rs[:, 4:5]

    latent = latent_ref[...]
    h = jnp.sin(W0_FIRST * (jnp.dot(latent, w0_ref[...],
                                    preferred_element_type=jnp.float32)
                            + bias_ref[0:1, :]))
    for i, w_ref in enumerate((w1_ref, w2_ref, w3_ref)):
        h = h + jnp.sin(jnp.dot(h, w_ref[...],
                                preferred_element_type=jnp.float32)
                        + bias_ref[i + 1:i + 2, :])

    val = (jnp.dot(h, wd_ref[...],
                   preferred_element_type=jnp.float32)
           + aux_ref[4:5, :] + aux_ref[3:4, :])

    gmat = jnp.concatenate(
        [jnp.concatenate([r00, r01, r02], axis=1),
         jnp.concatenate([r10, r11, r12], axis=1)], axis=0)
    pq = jnp.dot(gmat, aux_ref[0:3, :], preferred_element_type=jnp.float32)
    px = (pq[0:BATCH, :] + sx) + XSIZE / 2.0
    py = (pq[BATCH:2 * BATCH, :] + sy) + XSIZE / 2.0

    x0 = jnp.floor(px)
    y0 = jnp.floor(py)
    fx = px - x0
    fy = py - y0
    x0i = jnp.clip(x0.astype(jnp.int32), 0, XSIZE - 1)
    y0i = jnp.clip(y0.astype(jnp.int32), 0, XSIZE - 1)
    w00 = (1.0 - fx) * (1.0 - fy)
    w01 = fx * (1.0 - fy)
    w10 = (1.0 - fx) * fy
    w11 = fx * fy
    dx = (x0i < XSIZE - 1).astype(jnp.float32)
    dy = (y0i < XSIZE - 1).astype(jnp.float32)
    ex = 1.0 - dx
    ey = 1.0 - dy
    wa = w00 + ex * w01 + ey * w10 + ex * ey * w11
    wb = dx * (w01 + ey * w11)
    wc = dy * (w10 + ex * w11)
    wd_w = dx * dy * w11

    ii = y0i * XSIZE + x0i
    taps = [val * wa, val * wb, val * wc, val * wd_w]

    pos = lax.broadcasted_iota(jnp.int32, ii.shape, 1) & 3
    taken = [t for t in taps]
    is_dup = None
    for s in (1, 2, 3):
        nxt = jnp.roll(ii, -s, axis=1)
        prv = jnp.roll(ii, s, axis=1)
        m_add = (pos <= 3 - s) & (ii == nxt)
        m_dup = (pos >= s) & (ii == prv)
        is_dup = m_dup if is_dup is None else (is_dup | m_dup)
        zf = m_add.astype(jnp.float32)
        taken = [t + zf * jnp.roll(tap, -s, axis=1)
                 for t, tap in zip(taken, taps)]

    ii_eff = jnp.where(is_dup, NPIX + 8, ii)
    copy_off = (lax.broadcasted_iota(jnp.int32, ii.shape, 1) >> 2) & 3

    idx_ref[...] = ii_eff + copy_off * ACC
    va_ref[...] = jnp.where(is_dup, 0.0, taken[0])
    vb_ref[...] = jnp.where(is_dup, 0.0, taken[1])
    vc_ref[...] = jnp.where(is_dup, 0.0, taken[2])
    vd_ref[...] = jnp.where(is_dup, 0.0, taken[3])


def _stage_a(rs, latent, W0, W1, W2, W3, bias, aux, wd):
    nblk = VPAD // VB
    grid = (nblk,)
    fixed = lambda i: (0, 0)
    out_sd = jax.ShapeDtypeStruct((BATCH, VPAD), jnp.float32)
    return pl.pallas_call(
        _stage_a_body,
        grid=grid,
        in_specs=[
            pl.BlockSpec((BATCH, 8), fixed),
            pl.BlockSpec((BATCH, 64), fixed),
            pl.BlockSpec((64, 64), fixed),
            pl.BlockSpec((64, 64), fixed),
            pl.BlockSpec((64, 64), fixed),
            pl.BlockSpec((64, 64), fixed),
            pl.BlockSpec((8, 64), fixed),
            pl.BlockSpec((8, VB), lambda i: (0, i)),
            pl.BlockSpec((64, VB), lambda i: (0, i)),
        ],
        out_specs=[pl.BlockSpec((BATCH, VB), lambda i: (0, i))] * 5,
        out_shape=[jax.ShapeDtypeStruct((BATCH, VPAD), jnp.int32),
                   out_sd, out_sd, out_sd, out_sd],
    )(rs, latent, W0, W1, W2, W3, bias, aux, wd)


def _sc_scatter_body(idx_hbm, va_hbm, vb_hbm, vc_hbm, vd_hbm, out_hbm,
                     i0, i1, a0, a1, b0, b1, c0, c1, d0, d1,
                     acc, sem0, sem1):
    cid = lax.axis_index("c")
    sid = lax.axis_index("s")
    wid = sid * 2 + cid
    base = sid * VPAD + cid * HALF

    def zbody(i, _):
        acc[pl.ds(i * 16, 16)] = jnp.zeros((16,), jnp.float32)
        return 0
    lax.fori_loop(0, 4 * ACC // 16, zbody, 0)

    bufs = ((i0, a0, b0, c0, d0, sem0), (i1, a1, b1, c1, d1, sem1))

    def start(slot, c):
        off = base + c * CH
        ib, ab, bb, cb2, db, sem = bufs[slot]
        return [
            pltpu.async_copy(idx_hbm.at[pl.ds(off, CH)], ib, sem),
            pltpu.async_copy(va_hbm.at[pl.ds(off, CH)], ab, sem),
            pltpu.async_copy(vb_hbm.at[pl.ds(off, CH)], bb, sem),
            pltpu.async_copy(vc_hbm.at[pl.ds(off, CH)], cb2, sem),
            pltpu.async_copy(vd_hbm.at[pl.ds(off, CH)], db, sem),
        ]

    def consume(slot):
        ib, ab, bb, cb2, db, _ = bufs[slot]

        def cbody(i, _):
            o = i * 16
            iv = ib[pl.ds(o, 16)]
            va = ab[pl.ds(o, 16)]
            vb = bb[pl.ds(o, 16)]
            vc = cb2[pl.ds(o, 16)]
            vd = db[pl.ds(o, 16)]
            plsc.addupdate_scatter(acc, [iv], va)
            plsc.addupdate_scatter(acc, [iv + 1], vb)
            plsc.addupdate_scatter(acc, [iv + XSIZE], vc)
            plsc.addupdate_scatter(acc, [iv + (XSIZE + 1)], vd)
            return 0
        lax.fori_loop(0, CH // 16, cbody, 0)

    pend = start(0, 0)
    for c in range(NCHUNK):
        for cp in pend:
            cp.wait()
        nxt = start((c + 1) & 1, c + 1) if c + 1 < NCHUNK else None
        consume(c & 1)
        pend = nxt

    def mbody(i, _):
        o = i * 16
        acc[pl.ds(o, 16)] = (acc[pl.ds(o, 16)] + acc[pl.ds(o + ACC, 16)]
                             + acc[pl.ds(o + 2 * ACC, 16)]
                             + acc[pl.ds(o + 3 * ACC, 16)])
        return 0
    lax.fori_loop(0, NPIX // 16, mbody, 0)

    pltpu.sync_copy(acc.at[pl.ds(0, NPIX)], out_hbm.at[wid])


def _sc_scatter(idx_flat, va_flat, vb_flat, vc_flat, vd_flat):
    mesh = plsc.VectorSubcoreMesh(core_axis_name="c", subcore_axis_name="s")
    f = functools.partial(
        pl.kernel,
        out_type=jax.ShapeDtypeStruct((NW, NPIX), jnp.float32),
        mesh=mesh,
        compiler_params=pltpu.CompilerParams(needs_layout_passes=False),
        scratch_types=[
            pltpu.VMEM((CH,), jnp.int32), pltpu.VMEM((CH,), jnp.int32),
            pltpu.VMEM((CH,), jnp.float32), pltpu.VMEM((CH,), jnp.float32),
            pltpu.VMEM((CH,), jnp.float32), pltpu.VMEM((CH,), jnp.float32),
            pltpu.VMEM((CH,), jnp.float32), pltpu.VMEM((CH,), jnp.float32),
            pltpu.VMEM((CH,), jnp.float32), pltpu.VMEM((CH,), jnp.float32),
            pltpu.VMEM((4 * ACC,), jnp.float32),
            pltpu.SemaphoreType.DMA, pltpu.SemaphoreType.DMA,
        ],
    )(_sc_scatter_body)
    return f(idx_flat, va_flat, vb_flat, vc_flat, vd_flat)


def _filter_consts():
    t = np.arange(-3, 4, dtype=np.float64)
    k = np.exp(-0.5 * t * t)
    k = k / k.sum()
    kg = np.zeros((XSIZE, XSIZE), dtype=np.float64)
    for off, kv in zip(range(-3, 4), k):
        kg += np.diag(np.full(XSIZE - abs(off), kv), off)
    n = XSIZE
    u = np.arange(n)
    theta = 2.0 * np.pi * np.outer(u, u) / n
    F = np.exp(-1j * theta)
    A = F @ kg
    B = kg @ F
    C2 = np.conj(F) / (n * n)
    D2 = np.conj(F)
    def f32(x):
        return jnp.asarray(np.ascontiguousarray(x), dtype=jnp.float32)
    return (f32(A.real), f32(A.imag), f32(B.real), f32(B.imag),
            f32(C2.real), f32(C2.imag), f32(D2.real), f32(D2.imag))


def _stage_c_body(p_ref, ar_ref, ai_ref, br_ref, bi_ref,
                  cr_ref, ci_ref, dr_ref, di_ref, ctf_ref, out_ref):
    x = p_ref[0, 0] + p_ref[0, 1]
    dot = functools.partial(jnp.dot, precision=_HIGH,
                            preferred_element_type=jnp.float32)
    p = dot(ar_ref[...], x)
    q = dot(ai_ref[...], x)
    zr = dot(p, br_ref[...]) - dot(q, bi_ref[...])
    zi = dot(p, bi_ref[...]) + dot(q, br_ref[...])
    ctf = ctf_ref[...]
    zr = zr * ctf
    zi = zi * ctf
    u = dot(cr_ref[...], zr) - dot(ci_ref[...], zi)
    v = dot(cr_ref[...], zi) + dot(ci_ref[...], zr)
    out_ref[0] = dot(u, dr_ref[...]) - dot(v, di_ref[...])


def _stage_c(parts, ctf):
    consts = _filter_consts()
    fixed2 = lambda i: (0, 0)
    return pl.pallas_call(
        _stage_c_body,
        grid=(BATCH,),
        in_specs=[pl.BlockSpec((1, 2, XSIZE, XSIZE), lambda i: (i, 0, 0, 0))]
                 + [pl.BlockSpec((XSIZE, XSIZE), fixed2)] * 9,
        out_specs=pl.BlockSpec((1, XSIZE, XSIZE), lambda i: (i, 0, 0)),
        out_shape=jax.ShapeDtypeStruct((BATCH, XSIZE, XSIZE), jnp.float32),
    )(parts, *consts, ctf)


def kernel(rows, shifts, latent, coords, base_values,
           W0, b0, W1, b1, W2, b2, W3, b3, Wd, bd, ctf):
    v = coords.shape[0]
    pad = VPAD - v

    rs = jnp.zeros((BATCH, 8), jnp.float32)
    rs = rs.at[:, 0:3].set(rows).at[:, 3:5].set(shifts)
    bias = jnp.stack([b0, b1, b2, b3] + [jnp.zeros_like(b0)] * 4)

    aux = jnp.concatenate(
        [coords.T, base_values[None, :], bd[None, :],
         jnp.zeros((3, v), jnp.float32)], axis=0)
    aux = jnp.pad(aux, ((0, 0), (0, pad)))
    wd_p = jnp.pad(Wd, ((0, 0), (0, pad)))

    idx, va, vb, vc, vd = _stage_a(rs, latent, W0, W1, W2, W3, bias, aux, wd_p)

    parts = _sc_scatter(idx.reshape(-1), va.reshape(-1), vb.reshape(-1),
                        vc.reshape(-1), vd.reshape(-1))

    parts = parts.reshape(BATCH, 2, XSIZE, XSIZE)
    return _stage_c(parts, ctf)
